# baseline (speedup 1.0000x reference)
import functools

import jax
import jax.numpy as jnp
from jax import lax
from jax.experimental import pallas as pl
from jax.experimental.pallas import tpu as pltpu
from jax.experimental.pallas import tpu_sc as plsc

N = 10000
D = 128
E = 320000

NC = 2
NS = 16
NW = NC * NS
LANE = 16

CHUNK = 128
NCHUNKS = 80
EPW = NCHUNKS * CHUNK
EPAD = EPW * NW
NPAIRS = NCHUNKS // 2

N_A = 10240
NPT = N_A // NS
ZROWS = 128

_VMESH = plsc.VectorSubcoreMesh(core_axis_name="c", subcore_axis_name="s")


def _worker_id():
    return lax.axis_index("c") * NS + lax.axis_index("s")


@functools.partial(
    pl.kernel,
    out_type=jax.ShapeDtypeStruct((NW * N,), jnp.float32),
    mesh=_VMESH,
    compiler_params=pltpu.CompilerParams(needs_layout_passes=False),
    scratch_types=[
        pltpu.VMEM((CHUNK,), jnp.int32),
        pltpu.VMEM((CHUNK,), jnp.int32),
        pltpu.VMEM((CHUNK,), jnp.float32),
        pltpu.VMEM((N,), jnp.float32),
    ],
)
def _deg_partials(row_ref, col_ref, ew_ref, out_ref, rbuf, cbuf, wbuf, degl):
    wid = _worker_id()
    zeros16 = jnp.zeros((LANE,), jnp.float32)

    def zero_body(i, carry):
        degl[pl.ds(i * LANE, LANE)] = zeros16
        return carry

    lax.fori_loop(0, N // LANE, zero_body, 0)

    def chunk_body(k, carry):
        base = wid * EPW + k * CHUNK
        pltpu.sync_copy(row_ref.at[pl.ds(base, CHUNK)], rbuf)
        pltpu.sync_copy(col_ref.at[pl.ds(base, CHUNK)], cbuf)
        pltpu.sync_copy(ew_ref.at[pl.ds(base, CHUNK)], wbuf)
        for j in range(CHUNK // LANE):
            r = rbuf[pl.ds(j * LANE, LANE)]
            c = cbuf[pl.ds(j * LANE, LANE)]
            w = wbuf[pl.ds(j * LANE, LANE)]
            w = jnp.where(r == c, 0.0, w)
            plsc.addupdate_scatter(degl, [r], w)
        return carry

    lax.fori_loop(0, NCHUNKS, chunk_body, 0)
    pltpu.sync_copy(degl, out_ref.at[pl.ds(wid * N, N)])


def _dinv_body(p_ref, o_ref):
    deg = jnp.sum(p_ref[...], axis=0, keepdims=True)
    safe = jnp.where(deg > 0, deg, 1.0)
    o_ref[...] = jnp.where(deg > 0, lax.rsqrt(safe), 0.0)


def _dinv(partials):
    return pl.pallas_call(
        _dinv_body,
        out_shape=jax.ShapeDtypeStruct((1, N), jnp.float32),
    )(partials)


@functools.partial(
    pl.kernel,
    out_type=jax.ShapeDtypeStruct((EPAD,), jnp.float32),
    mesh=_VMESH,
    compiler_params=pltpu.CompilerParams(needs_layout_passes=False),
    scratch_types=[
        pltpu.VMEM((CHUNK,), jnp.int32),
        pltpu.VMEM((CHUNK,), jnp.int32),
        pltpu.VMEM((CHUNK,), jnp.float32),
        pltpu.VMEM((CHUNK,), jnp.float32),
        pltpu.VMEM((N,), jnp.float32),
    ],
)
def _lap_weights(row_ref, col_ref, ew_ref, dinv_ref, out_ref,
                 rbuf, cbuf, wbuf, lwbuf, dinv_v):
    wid = _worker_id()
    pltpu.sync_copy(dinv_ref, dinv_v)

    def chunk_body(k, carry):
        base = wid * EPW + k * CHUNK
        pltpu.sync_copy(row_ref.at[pl.ds(base, CHUNK)], rbuf)
        pltpu.sync_copy(col_ref.at[pl.ds(base, CHUNK)], cbuf)
        pltpu.sync_copy(ew_ref.at[pl.ds(base, CHUNK)], wbuf)
        for j in range(CHUNK // LANE):
            r = rbuf[pl.ds(j * LANE, LANE)]
            c = cbuf[pl.ds(j * LANE, LANE)]
            w = wbuf[pl.ds(j * LANE, LANE)]
            dr = plsc.load_gather(dinv_v, [r])
            dc = plsc.load_gather(dinv_v, [c])
            lw = -(dr * w * dc)
            lw = jnp.where(r == c, 0.0, lw)
            lwbuf[pl.ds(j * LANE, LANE)] = lw
        pltpu.sync_copy(lwbuf, out_ref.at[pl.ds(base, CHUNK)])
        return carry

    lax.fori_loop(0, NCHUNKS, chunk_body, 0)


PCHUNK = 64
PNCH = EPW // PCHUNK
NQUADS = PNCH // 4
ISLOTS = 8


@functools.partial(
    pl.kernel,
    out_type=jax.ShapeDtypeStruct((NC * N_A, D), jnp.float32),
    mesh=_VMESH,
    compiler_params=pltpu.CompilerParams(
        needs_layout_passes=False, use_tc_tiling_on_sc=False),
    scratch_types=[
        pltpu.VMEM((ISLOTS, PCHUNK), jnp.int32),
        pltpu.VMEM((ISLOTS, PCHUNK), jnp.int32),
        pltpu.VMEM((ISLOTS, PCHUNK), jnp.float32),
        pltpu.VMEM((PCHUNK, D // 2), jnp.int32),
        pltpu.VMEM((PCHUNK, D // 2), jnp.int32),
        pltpu.VMEM((PCHUNK, D), jnp.float32),
        pltpu.VMEM((PCHUNK, D), jnp.float32),
        pltpu.VMEM_SHARED((N_A, D), jnp.float32),
        pltpu.SemaphoreType.DMA,
        pltpu.SemaphoreType.DMA,
        pltpu.SemaphoreType.DMA,
        pltpu.SemaphoreType.DMA,
        pltpu.SemaphoreType.DMA,
        pltpu.SemaphoreType.DMA,
        pltpu.SemaphoreType.DMA,
        pltpu.SemaphoreType.DMA,
    ],
)
def _prop(row_ref, col_ref, lw_ref, h_ref, z_ref, out_ref,
          rowb, colb, wb, rg0, rg1, rs0, rs1, acc,
          isem0, isem1, isem2, isem3, gsem0, gsem1, ssem0, ssem1):
    cid = lax.axis_index("c")
    sid = lax.axis_index("s")
    wid = cid * NS + sid
    isems = (isem0, isem1, isem2, isem3)
    base = wid * EPW
    r0 = sid * NPT

    def idx_descs(k, isem):
        slot = lax.rem(k, ISLOTS)
        src = pl.ds(base + k * PCHUNK, PCHUNK)
        return (
            pltpu.make_async_copy(row_ref.at[src], rowb.at[slot], isem),
            pltpu.make_async_copy(col_ref.at[src], colb.at[slot], isem),
            pltpu.make_async_copy(lw_ref.at[src], wb.at[slot], isem),
        )

    def idx_start(k, isem):
        for d in idx_descs(k, isem):
            d.start()

    def idx_wait(k, isem):
        for d in idx_descs(k, isem):
            d.wait()

    def gather_desc(k, rg, gsem):
        return pltpu.make_async_copy(
            h_ref.at[rowb.at[lax.rem(k, ISLOTS)]], rg, gsem)

    def scatter_desc(k, rs, ssem):
        return pltpu.make_async_copy(
            rs, acc.at[colb.at[lax.rem(k, ISLOTS)]], ssem)

    def scale(k, rg, rs):
        slot_splat = jnp.zeros((LANE,), jnp.int32) + lax.rem(k, ISLOTS)
        himask = jnp.full((LANE,), -65536, jnp.int32)

        def srow(j, c2):
            wv = plsc.load_gather(
                wb, [slot_splat, jnp.zeros((LANE,), jnp.int32) + j])
            for q in range(D // 32):
                v = rg[j, pl.ds(q * LANE, LANE)]
                lo = lax.bitcast_convert_type(
                    lax.shift_left(v, 16), jnp.float32)
                hi = lax.bitcast_convert_type(v & himask, jnp.float32)
                rs[j, pl.ds(2 * q * LANE, LANE)] = lo * wv
                rs[j, pl.ds((2 * q + 1) * LANE, LANE)] = hi * wv
            return c2

        lax.fori_loop(0, PCHUNK, srow, 0)

    idx_start(0, isems[0])
    idx_start(1, isems[1])
    idx_start(2, isems[2])
    idx_start(3, isems[3])
    pltpu.sync_copy(z_ref, acc.at[pl.ds(r0, NPT)])
    plsc.subcore_barrier()
    idx_wait(0, isems[0])
    gather_desc(0, rg0, gsem0).start()
    idx_start(4, isems[0])
    idx_wait(1, isems[1])
    gather_desc(1, rg1, gsem1).start()
    idx_start(5, isems[1])

    def slot_step(k, t, rg, rs, gsem, ssem):
        isem_n = isems[(t + 2) % 4]
        gather_desc(k, rg, gsem).wait()

        @pl.when(k >= 2)
        def _():
            scatter_desc(k - 2, rs, ssem).wait()

        scale(k, rg, rs)

        @pl.when(k + 2 < PNCH)
        def _():
            idx_wait(k + 2, isem_n)
            gather_desc(k + 2, rg, gsem).start()

        scatter_desc(k, rs, ssem).start(add=True)

        @pl.when(k + 6 < PNCH)
        def _():
            idx_start(k + 6, isem_n)

    def quad(q, carry):
        k = 4 * q
        slot_step(k, 0, rg0, rs0, gsem0, ssem0)
        slot_step(k + 1, 1, rg1, rs1, gsem1, ssem1)
        slot_step(k + 2, 2, rg0, rs0, gsem0, ssem0)
        slot_step(k + 3, 3, rg1, rs1, gsem1, ssem1)
        return carry

    lax.fori_loop(0, NQUADS, quad, 0)
    scatter_desc(PNCH - 2, rs0, ssem0).wait()
    scatter_desc(PNCH - 1, rs1, ssem1).wait()
    plsc.subcore_barrier()
    pltpu.sync_copy(acc.at[pl.ds(r0, NPT)],
                    out_ref.at[pl.ds(cid * N_A + r0, NPT)])


BN = 1000


def _round_bf16_bits(x):
    u = lax.bitcast_convert_type(x, jnp.int32)
    rb = lax.shift_right_logical(u, 16) & 1
    return lax.shift_right_logical(u + 32767 + rb, 16)


def _pack_rows(h):
    bn = h.shape[0]
    h4 = h.reshape(bn, D // 32, 2, LANE)
    lo = _round_bf16_bits(h4[:, :, 0, :])
    hi = _round_bf16_bits(h4[:, :, 1, :])
    return (lo | lax.shift_left(hi, 16)).reshape(bn, D // 2)


def _combine_body(p_ref, o_ref, ob_ref):
    tx1 = p_ref[0] + p_ref[1]
    o_ref[...] = tx1
    ob_ref[...] = _pack_rows(tx1)


def _combine(p):
    return pl.pallas_call(
        _combine_body,
        grid=(N // BN,),
        in_specs=[pl.BlockSpec((NC, BN, D), lambda i: (0, i, 0))],
        out_specs=[pl.BlockSpec((BN, D), lambda i: (i, 0)),
                   pl.BlockSpec((BN, D // 2), lambda i: (i, 0))],
        out_shape=[jax.ShapeDtypeStruct((N, D), jnp.float32),
                   jax.ShapeDtypeStruct((N, D // 2), jnp.int32)],
    )(p)


def _pack_body(h_ref, ob_ref):
    ob_ref[...] = _pack_rows(h_ref[...])


def _pack(h):
    return pl.pallas_call(
        _pack_body,
        grid=(N // BN,),
        in_specs=[pl.BlockSpec((BN, D), lambda i: (i, 0))],
        out_specs=pl.BlockSpec((BN, D // 2), lambda i: (i, 0)),
        out_shape=jax.ShapeDtypeStruct((N, D // 2), jnp.int32),
    )(h)


def _dense_body(tx0_ref, tx1_ref, q_ref, w_ref, b_ref, o_ref, ob_ref):
    tx0 = tx0_ref[...]
    tx1 = tx1_ref[...]
    tx2 = 2.0 * (q_ref[0] + q_ref[1]) - tx0
    acc = jnp.dot(tx0, w_ref[0], preferred_element_type=jnp.float32)
    acc = acc + jnp.dot(tx1, w_ref[1], preferred_element_type=jnp.float32)
    acc = acc + jnp.dot(tx2, w_ref[2], preferred_element_type=jnp.float32)
    h = jax.nn.sigmoid(acc + b_ref[...])
    o_ref[...] = h
    ob_ref[...] = _pack_rows(h)


def _dense(tx0, tx1, q, w, b):
    return pl.pallas_call(
        _dense_body,
        grid=(N // BN,),
        in_specs=[
            pl.BlockSpec((BN, D), lambda i: (i, 0)),
            pl.BlockSpec((BN, D), lambda i: (i, 0)),
            pl.BlockSpec((NC, BN, D), lambda i: (0, i, 0)),
            pl.BlockSpec((3, D, D), lambda i: (0, 0, 0)),
            pl.BlockSpec((1, D), lambda i: (0, 0)),
        ],
        out_specs=[pl.BlockSpec((BN, D), lambda i: (i, 0)),
                   pl.BlockSpec((BN, D // 2), lambda i: (i, 0))],
        out_shape=[jax.ShapeDtypeStruct((N, D), jnp.float32),
                   jax.ShapeDtypeStruct((N, D // 2), jnp.int32)],
    )(tx0, tx1, q, w, b)


def kernel(x, edge_index, edge_weight, W1, b1, W2, b2, W3, b3, W4, b4):
    perm = jnp.argsort(edge_index[1])
    row = edge_index[0][perm]
    col = edge_index[1][perm]
    edge_weight = edge_weight[perm]
    pad = EPAD - E
    row_p = jnp.pad(row, (0, pad))
    col_p = jnp.pad(col, (0, pad))
    ew_p = jnp.pad(edge_weight, (0, pad))

    partials = _deg_partials(row_p, col_p, ew_p).reshape(NW, N)
    dinv = _dinv(partials).reshape(N)
    lw = _lap_weights(row_p, col_p, ew_p, dinv)

    z = jnp.zeros((NPT, D), jnp.float32)
    h = x
    hp = _pack(x)
    for w, b in ((W1, b1), (W2, b2), (W3, b3), (W4, b4)):
        p = _prop(row_p, col_p, lw, hp, z).reshape(NC, N_A, D)
        tx1, tx1p = _combine(p)
        q = _prop(row_p, col_p, lw, tx1p, z).reshape(NC, N_A, D)
        h, hp = _dense(h, tx1, q, w, b.reshape(1, D))
    return h


# speedup vs baseline: 1.0366x; 1.0366x over previous
import functools

import jax
import jax.numpy as jnp
from jax import lax
from jax.experimental import pallas as pl
from jax.experimental.pallas import tpu as pltpu
from jax.experimental.pallas import tpu_sc as plsc

N = 10000
D = 128
E = 320000

NC = 2
NS = 16
NW = NC * NS
LANE = 16

CHUNK = 128
NCHUNKS = 80
EPW = NCHUNKS * CHUNK
EPAD = EPW * NW
NPAIRS = NCHUNKS // 2

N_A = 10240
NPT = N_A // NS
ZROWS = 128

_VMESH = plsc.VectorSubcoreMesh(core_axis_name="c", subcore_axis_name="s")


def _worker_id():
    return lax.axis_index("c") * NS + lax.axis_index("s")


@functools.partial(
    pl.kernel,
    out_type=jax.ShapeDtypeStruct((NW * N,), jnp.float32),
    mesh=_VMESH,
    compiler_params=pltpu.CompilerParams(needs_layout_passes=False),
    scratch_types=[
        pltpu.VMEM((CHUNK,), jnp.int32),
        pltpu.VMEM((CHUNK,), jnp.int32),
        pltpu.VMEM((CHUNK,), jnp.float32),
        pltpu.VMEM((N,), jnp.float32),
    ],
)
def _deg_partials(row_ref, col_ref, ew_ref, out_ref, rbuf, cbuf, wbuf, degl):
    wid = _worker_id()
    zeros16 = jnp.zeros((LANE,), jnp.float32)

    def zero_body(i, carry):
        degl[pl.ds(i * LANE, LANE)] = zeros16
        return carry

    lax.fori_loop(0, N // LANE, zero_body, 0)

    def chunk_body(k, carry):
        base = wid * EPW + k * CHUNK
        pltpu.sync_copy(row_ref.at[pl.ds(base, CHUNK)], rbuf)
        pltpu.sync_copy(col_ref.at[pl.ds(base, CHUNK)], cbuf)
        pltpu.sync_copy(ew_ref.at[pl.ds(base, CHUNK)], wbuf)
        for j in range(CHUNK // LANE):
            r = rbuf[pl.ds(j * LANE, LANE)]
            c = cbuf[pl.ds(j * LANE, LANE)]
            w = wbuf[pl.ds(j * LANE, LANE)]
            w = jnp.where(r == c, 0.0, w)
            plsc.addupdate_scatter(degl, [r], w)
        return carry

    lax.fori_loop(0, NCHUNKS, chunk_body, 0)
    pltpu.sync_copy(degl, out_ref.at[pl.ds(wid * N, N)])


def _dinv_body(p_ref, o_ref):
    deg = jnp.sum(p_ref[...], axis=0, keepdims=True)
    safe = jnp.where(deg > 0, deg, 1.0)
    o_ref[...] = jnp.where(deg > 0, lax.rsqrt(safe), 0.0)


def _dinv(partials):
    return pl.pallas_call(
        _dinv_body,
        out_shape=jax.ShapeDtypeStruct((1, N), jnp.float32),
    )(partials)


@functools.partial(
    pl.kernel,
    out_type=jax.ShapeDtypeStruct((EPAD,), jnp.float32),
    mesh=_VMESH,
    compiler_params=pltpu.CompilerParams(needs_layout_passes=False),
    scratch_types=[
        pltpu.VMEM((CHUNK,), jnp.int32),
        pltpu.VMEM((CHUNK,), jnp.int32),
        pltpu.VMEM((CHUNK,), jnp.float32),
        pltpu.VMEM((CHUNK,), jnp.float32),
        pltpu.VMEM((N,), jnp.float32),
    ],
)
def _lap_weights(row_ref, col_ref, ew_ref, dinv_ref, out_ref,
                 rbuf, cbuf, wbuf, lwbuf, dinv_v):
    wid = _worker_id()
    pltpu.sync_copy(dinv_ref, dinv_v)

    def chunk_body(k, carry):
        base = wid * EPW + k * CHUNK
        pltpu.sync_copy(row_ref.at[pl.ds(base, CHUNK)], rbuf)
        pltpu.sync_copy(col_ref.at[pl.ds(base, CHUNK)], cbuf)
        pltpu.sync_copy(ew_ref.at[pl.ds(base, CHUNK)], wbuf)
        for j in range(CHUNK // LANE):
            r = rbuf[pl.ds(j * LANE, LANE)]
            c = cbuf[pl.ds(j * LANE, LANE)]
            w = wbuf[pl.ds(j * LANE, LANE)]
            dr = plsc.load_gather(dinv_v, [r])
            dc = plsc.load_gather(dinv_v, [c])
            lw = -(dr * w * dc)
            lw = jnp.where(r == c, 0.0, lw)
            lwbuf[pl.ds(j * LANE, LANE)] = lw
        pltpu.sync_copy(lwbuf, out_ref.at[pl.ds(base, CHUNK)])
        return carry

    lax.fori_loop(0, NCHUNKS, chunk_body, 0)


PCHUNK = 64
ISLOTS = 4
RPW = N_A // NW
ACC_ROWS = RPW + 8
DUMP = RPW


@functools.partial(
    pl.kernel,
    out_type=jax.ShapeDtypeStruct((N_A, D), jnp.float32),
    mesh=_VMESH,
    compiler_params=pltpu.CompilerParams(
        needs_layout_passes=False, use_tc_tiling_on_sc=False),
    scratch_types=[
        pltpu.VMEM((ISLOTS, PCHUNK), jnp.int32),
        pltpu.VMEM((ISLOTS, PCHUNK), jnp.int32),
        pltpu.VMEM((ISLOTS, PCHUNK), jnp.float32),
        pltpu.VMEM((ISLOTS, PCHUNK), jnp.int32),
        pltpu.VMEM((PCHUNK, D // 2), jnp.int32),
        pltpu.VMEM((PCHUNK, D // 2), jnp.int32),
        pltpu.VMEM((ACC_ROWS, D), jnp.float32),
        pltpu.VMEM((LANE,), jnp.int32),
        pltpu.SemaphoreType.DMA,
        pltpu.SemaphoreType.DMA,
        pltpu.SemaphoreType.DMA,
        pltpu.SemaphoreType.DMA,
    ],
)
def _prop(row_ref, col_ref, lw_ref, h_ref, par_ref, out_ref,
          rowb, colb, wbf, colloc, rg0, rg1, acc, pvb,
          isem0, isem1, gsem0, gsem1):
    cid = lax.axis_index("c")
    sid = lax.axis_index("s")
    wid = cid * NS + sid
    zeros16 = jnp.zeros((LANE,), jnp.float32)
    iota16 = lax.iota(jnp.int32, LANE)
    himask = jnp.full((LANE,), -65536, jnp.int32)

    pltpu.sync_copy(par_ref.at[pl.ds(pl.multiple_of(wid * LANE, 8), LANE)], pvb)
    pv = pvb[pl.ds(0, LANE)]
    o_t = pv[0]
    e_t = pv[1]
    s_t = pv[2]
    npairs = pv[3]
    base = pv[4]
    nch = 2 * npairs

    def zbody(i, carry):
        for qq in range(D // LANE):
            acc[i, pl.ds(qq * LANE, LANE)] = zeros16
        return carry

    lax.fori_loop(0, ACC_ROWS, zbody, 0)

    def idx_descs(k, isem):
        slot = lax.rem(k, ISLOTS)
        srcs = pl.ds(pl.multiple_of(s_t + k * PCHUNK, 8), PCHUNK)
        return (
            pltpu.make_async_copy(row_ref.at[srcs], rowb.at[slot], isem),
            pltpu.make_async_copy(col_ref.at[srcs], colb.at[slot], isem),
            pltpu.make_async_copy(lw_ref.at[srcs], wbf.at[slot], isem),
        )

    def idx_start(k, isem):
        for d in idx_descs(k, isem):
            d.start()

    def idx_wait(k, isem):
        for d in idx_descs(k, isem):
            d.wait()

    def gather_desc(k, rg, gsem):
        return pltpu.make_async_copy(
            h_ref.at[rowb.at[lax.rem(k, ISLOTS)]], rg, gsem)

    idx_start(0, isem0)
    idx_start(1, isem1)
    idx_wait(0, isem0)
    gather_desc(0, rg0, gsem0).start()
    idx_start(2, isem0)
    idx_wait(1, isem1)
    gather_desc(1, rg1, gsem1).start()
    idx_start(3, isem1)

    def slot_step(k, rg, gsem, isem):
        sl = lax.rem(k, ISLOTS)
        gather_desc(k, rg, gsem).wait()

        jg0 = s_t + k * PCHUNK
        for i in range(PCHUNK // LANE):
            c = colb[sl, pl.ds(i * LANE, LANE)]
            jg = jg0 + i * LANE + iota16
            valid = (jg >= o_t) & (jg < e_t)
            colloc[sl, pl.ds(i * LANE, LANE)] = jnp.where(valid, c - base, DUMP)

        sl_splat = jnp.zeros((LANE,), jnp.int32) + sl

        def edge(j, carry):
            j_splat = jnp.zeros((LANE,), jnp.int32) + j
            cvec = plsc.load_gather(colloc, [sl_splat, j_splat])
            wv = plsc.load_gather(wbf, [sl_splat, j_splat])
            for q in range(D // 32):
                v = rg[j, pl.ds(q * LANE, LANE)]
                lo = lax.bitcast_convert_type(
                    lax.shift_left(v, 16), jnp.float32)
                hi = lax.bitcast_convert_type(v & himask, jnp.float32)
                plsc.addupdate_scatter(
                    acc, [cvec, iota16 + (2 * q * LANE)], lo * wv)
                plsc.addupdate_scatter(
                    acc, [cvec, iota16 + ((2 * q + 1) * LANE)], hi * wv)
            return carry

        lax.fori_loop(0, PCHUNK, edge, 0)

        @pl.when(k + 2 < nch)
        def _():
            idx_wait(k + 2, isem)
            gather_desc(k + 2, rg, gsem).start()

        @pl.when(k + 4 < nch)
        def _():
            idx_start(k + 4, isem)

    def pair(m, carry):
        k = 2 * m
        slot_step(k, rg0, gsem0, isem0)
        slot_step(k + 1, rg1, gsem1, isem1)
        return carry

    lax.fori_loop(0, npairs, pair, 0)
    pltpu.sync_copy(acc.at[pl.ds(0, RPW)],
                    out_ref.at[pl.ds(pl.multiple_of(wid * RPW, 8), RPW)])


BN = 1000


def _round_bf16_bits(x):
    u = lax.bitcast_convert_type(x, jnp.int32)
    rb = lax.shift_right_logical(u, 16) & 1
    return lax.shift_right_logical(u + 32767 + rb, 16)


def _pack_rows(h):
    bn = h.shape[0]
    h4 = h.reshape(bn, D // 32, 2, LANE)
    lo = _round_bf16_bits(h4[:, :, 0, :])
    hi = _round_bf16_bits(h4[:, :, 1, :])
    return (lo | lax.shift_left(hi, 16)).reshape(bn, D // 2)


def _pack_body(h_ref, ob_ref):
    ob_ref[...] = _pack_rows(h_ref[...])


def _pack(h):
    return pl.pallas_call(
        _pack_body,
        grid=(N // BN,),
        in_specs=[pl.BlockSpec((BN, D), lambda i: (i, 0))],
        out_specs=pl.BlockSpec((BN, D // 2), lambda i: (i, 0)),
        out_shape=jax.ShapeDtypeStruct((N, D // 2), jnp.int32),
    )(h)


def _dense_body(tx0_ref, tx1_ref, q_ref, w_ref, b_ref, o_ref, ob_ref):
    tx0 = tx0_ref[...]
    tx1 = tx1_ref[...]
    tx2 = 2.0 * q_ref[...] - tx0
    acc = jnp.dot(tx0, w_ref[0], preferred_element_type=jnp.float32)
    acc = acc + jnp.dot(tx1, w_ref[1], preferred_element_type=jnp.float32)
    acc = acc + jnp.dot(tx2, w_ref[2], preferred_element_type=jnp.float32)
    h = jax.nn.sigmoid(acc + b_ref[...])
    o_ref[...] = h
    ob_ref[...] = _pack_rows(h)


def _dense(tx0, tx1, q, w, b):
    return pl.pallas_call(
        _dense_body,
        grid=(N // BN,),
        in_specs=[
            pl.BlockSpec((BN, D), lambda i: (i, 0)),
            pl.BlockSpec((BN, D), lambda i: (i, 0)),
            pl.BlockSpec((BN, D), lambda i: (i, 0)),
            pl.BlockSpec((3, D, D), lambda i: (0, 0, 0)),
            pl.BlockSpec((1, D), lambda i: (0, 0)),
        ],
        out_specs=[pl.BlockSpec((BN, D), lambda i: (i, 0)),
                   pl.BlockSpec((BN, D // 2), lambda i: (i, 0))],
        out_shape=[jax.ShapeDtypeStruct((N, D), jnp.float32),
                   jax.ShapeDtypeStruct((N, D // 2), jnp.int32)],
    )(tx0, tx1, q, w, b)


def kernel(x, edge_index, edge_weight, W1, b1, W2, b2, W3, b3, W4, b4):
    perm = jnp.argsort(edge_index[1])
    row = edge_index[0][perm]
    col = edge_index[1][perm]
    ew = edge_weight[perm]
    pad = EPAD - E
    row_p = jnp.pad(row, (0, pad))
    col_p = jnp.pad(col, (0, pad))
    ew_p = jnp.pad(ew, (0, pad))

    o = jnp.searchsorted(col, jnp.arange(NW + 1, dtype=jnp.int32) * RPW)
    o = o.astype(jnp.int32)
    o_t = o[:-1]
    e_t = o[1:]
    s_t = o_t - (o_t % 8)
    nch = (e_t - s_t + PCHUNK - 1) // PCHUNK
    npairs = jnp.maximum((nch + 1) // 2, 1)
    base = jnp.arange(NW, dtype=jnp.int32) * RPW
    zero_col = jnp.zeros((NW,), jnp.int32)
    params = jnp.stack(
        [o_t, e_t, s_t, npairs, base] + [zero_col] * 11, axis=1
    ).reshape(NW * LANE).astype(jnp.int32)

    partials = _deg_partials(row_p, col_p, ew_p).reshape(NW, N)
    dinv = _dinv(partials).reshape(N)
    lw = _lap_weights(row_p, col_p, ew_p, dinv)

    h = x
    hp = _pack(x)
    for w, b in ((W1, b1), (W2, b2), (W3, b3), (W4, b4)):
        p = _prop(row_p, col_p, lw, hp, params)
        tx1p = _pack(p)
        q = _prop(row_p, col_p, lw, tx1p, params)
        h, hp = _dense(h, p, q, w, b.reshape(1, D))
    return h


# speedup vs baseline: 1.6446x; 1.5865x over previous
import functools

import jax
import jax.numpy as jnp
from jax import lax
from jax.experimental import pallas as pl
from jax.experimental.pallas import tpu as pltpu
from jax.experimental.pallas import tpu_sc as plsc

N = 10000
D = 128
E = 320000

NC = 2
NS = 16
NW = NC * NS
LANE = 16

CHUNK = 128
NCHUNKS = 80
EPW = NCHUNKS * CHUNK
EPAD = EPW * NW
NPAIRS = NCHUNKS // 2

N_A = 10240
NPT = N_A // NS
ZROWS = 128

_VMESH = plsc.VectorSubcoreMesh(core_axis_name="c", subcore_axis_name="s")


def _worker_id():
    return lax.axis_index("c") * NS + lax.axis_index("s")


@functools.partial(
    pl.kernel,
    out_type=jax.ShapeDtypeStruct((NW * N,), jnp.float32),
    mesh=_VMESH,
    compiler_params=pltpu.CompilerParams(needs_layout_passes=False),
    scratch_types=[
        pltpu.VMEM((CHUNK,), jnp.int32),
        pltpu.VMEM((CHUNK,), jnp.int32),
        pltpu.VMEM((CHUNK,), jnp.float32),
        pltpu.VMEM((N,), jnp.float32),
    ],
)
def _deg_partials(row_ref, col_ref, ew_ref, out_ref, rbuf, cbuf, wbuf, degl):
    wid = _worker_id()
    zeros16 = jnp.zeros((LANE,), jnp.float32)

    def zero_body(i, carry):
        degl[pl.ds(i * LANE, LANE)] = zeros16
        return carry

    lax.fori_loop(0, N // LANE, zero_body, 0)

    def chunk_body(k, carry):
        base = wid * EPW + k * CHUNK
        pltpu.sync_copy(row_ref.at[pl.ds(base, CHUNK)], rbuf)
        pltpu.sync_copy(col_ref.at[pl.ds(base, CHUNK)], cbuf)
        pltpu.sync_copy(ew_ref.at[pl.ds(base, CHUNK)], wbuf)
        for j in range(CHUNK // LANE):
            r = rbuf[pl.ds(j * LANE, LANE)]
            c = cbuf[pl.ds(j * LANE, LANE)]
            w = wbuf[pl.ds(j * LANE, LANE)]
            w = jnp.where(r == c, 0.0, w)
            plsc.addupdate_scatter(degl, [r], w)
        return carry

    lax.fori_loop(0, NCHUNKS, chunk_body, 0)
    pltpu.sync_copy(degl, out_ref.at[pl.ds(wid * N, N)])


def _dinv_body(p_ref, o_ref):
    deg = jnp.sum(p_ref[...], axis=0, keepdims=True)
    safe = jnp.where(deg > 0, deg, 1.0)
    o_ref[...] = jnp.where(deg > 0, lax.rsqrt(safe), 0.0)


def _dinv(partials):
    return pl.pallas_call(
        _dinv_body,
        out_shape=jax.ShapeDtypeStruct((1, N), jnp.float32),
    )(partials)


@functools.partial(
    pl.kernel,
    out_type=jax.ShapeDtypeStruct((EPAD,), jnp.float32),
    mesh=_VMESH,
    compiler_params=pltpu.CompilerParams(needs_layout_passes=False),
    scratch_types=[
        pltpu.VMEM((CHUNK,), jnp.int32),
        pltpu.VMEM((CHUNK,), jnp.int32),
        pltpu.VMEM((CHUNK,), jnp.float32),
        pltpu.VMEM((CHUNK,), jnp.float32),
        pltpu.VMEM((N,), jnp.float32),
    ],
)
def _lap_weights(row_ref, col_ref, ew_ref, dinv_ref, out_ref,
                 rbuf, cbuf, wbuf, lwbuf, dinv_v):
    wid = _worker_id()
    pltpu.sync_copy(dinv_ref, dinv_v)

    def chunk_body(k, carry):
        base = wid * EPW + k * CHUNK
        pltpu.sync_copy(row_ref.at[pl.ds(base, CHUNK)], rbuf)
        pltpu.sync_copy(col_ref.at[pl.ds(base, CHUNK)], cbuf)
        pltpu.sync_copy(ew_ref.at[pl.ds(base, CHUNK)], wbuf)
        for j in range(CHUNK // LANE):
            r = rbuf[pl.ds(j * LANE, LANE)]
            c = cbuf[pl.ds(j * LANE, LANE)]
            w = wbuf[pl.ds(j * LANE, LANE)]
            dr = plsc.load_gather(dinv_v, [r])
            dc = plsc.load_gather(dinv_v, [c])
            lw = -(dr * w * dc)
            lw = jnp.where(r == c, 0.0, lw)
            lwbuf[pl.ds(j * LANE, LANE)] = lw
        pltpu.sync_copy(lwbuf, out_ref.at[pl.ds(base, CHUNK)])
        return carry

    lax.fori_loop(0, NCHUNKS, chunk_body, 0)


PCHUNK = 64
ISLOTS = 4
RPW = N_A // NW
ACC_ROWS = RPW + 8
DUMP = RPW


@functools.partial(
    pl.kernel,
    out_type=jax.ShapeDtypeStruct((N_A, D), jnp.float32),
    mesh=_VMESH,
    compiler_params=pltpu.CompilerParams(
        needs_layout_passes=False, use_tc_tiling_on_sc=False),
    scratch_types=[
        pltpu.VMEM((ISLOTS, PCHUNK), jnp.int32),
        pltpu.VMEM((ISLOTS, PCHUNK), jnp.int32),
        pltpu.VMEM((ISLOTS, PCHUNK), jnp.float32),
        pltpu.VMEM((ISLOTS, PCHUNK), jnp.int32),
        pltpu.VMEM((PCHUNK, D // 2), jnp.int32),
        pltpu.VMEM((PCHUNK, D // 2), jnp.int32),
        pltpu.VMEM((ACC_ROWS, D), jnp.float32),
        pltpu.VMEM((LANE,), jnp.int32),
        pltpu.SemaphoreType.DMA,
        pltpu.SemaphoreType.DMA,
        pltpu.SemaphoreType.DMA,
        pltpu.SemaphoreType.DMA,
    ],
)
def _prop(row_ref, col_ref, lw_ref, h_ref, par_ref, out_ref,
          rowb, colb, wbf, colloc, rg0, rg1, acc, pvb,
          isem0, isem1, gsem0, gsem1):
    cid = lax.axis_index("c")
    sid = lax.axis_index("s")
    wid = cid * NS + sid
    zeros16 = jnp.zeros((LANE,), jnp.float32)
    iota16 = lax.iota(jnp.int32, LANE)
    himask = jnp.full((LANE,), -65536, jnp.int32)

    pltpu.sync_copy(par_ref.at[pl.ds(pl.multiple_of(wid * LANE, 8), LANE)], pvb)
    pv = pvb[pl.ds(0, LANE)]
    o_t = pv[0]
    e_t = pv[1]
    s_t = pv[2]
    npairs = pv[3]
    base = pv[4]
    nch = 2 * npairs

    def zbody(i, carry):
        for qq in range(D // LANE):
            acc[i, pl.ds(qq * LANE, LANE)] = zeros16
        return carry

    lax.fori_loop(0, ACC_ROWS, zbody, 0)

    def idx_descs(k, isem):
        slot = lax.rem(k, ISLOTS)
        srcs = pl.ds(pl.multiple_of(s_t + k * PCHUNK, 8), PCHUNK)
        return (
            pltpu.make_async_copy(row_ref.at[srcs], rowb.at[slot], isem),
            pltpu.make_async_copy(col_ref.at[srcs], colb.at[slot], isem),
            pltpu.make_async_copy(lw_ref.at[srcs], wbf.at[slot], isem),
        )

    def idx_start(k, isem):
        for d in idx_descs(k, isem):
            d.start()

    def idx_wait(k, isem):
        for d in idx_descs(k, isem):
            d.wait()

    def gather_desc(k, rg, gsem):
        return pltpu.make_async_copy(
            h_ref.at[rowb.at[lax.rem(k, ISLOTS)]], rg, gsem)

    idx_start(0, isem0)
    idx_start(1, isem1)
    idx_wait(0, isem0)
    gather_desc(0, rg0, gsem0).start()
    idx_start(2, isem0)
    idx_wait(1, isem1)
    gather_desc(1, rg1, gsem1).start()
    idx_start(3, isem1)

    def slot_step(k, rg, gsem, isem):
        sl = lax.rem(k, ISLOTS)
        gather_desc(k, rg, gsem).wait()

        jg0 = s_t + k * PCHUNK
        for i in range(PCHUNK // LANE):
            c = colb[sl, pl.ds(i * LANE, LANE)]
            jg = jg0 + i * LANE + iota16
            valid = (jg >= o_t) & (jg < e_t)
            colloc[sl, pl.ds(i * LANE, LANE)] = jnp.where(valid, c - base, DUMP)

        sl_splat = jnp.zeros((LANE,), jnp.int32) + sl

        @plsc.parallel_loop(0, PCHUNK, 1, unroll=4)
        def _edge(j):
            j_splat = jnp.zeros((LANE,), jnp.int32) + j
            cvec = plsc.load_gather(colloc, [sl_splat, j_splat])
            wv = plsc.load_gather(wbf, [sl_splat, j_splat])
            for q in range(D // 32):
                v = rg[j, pl.ds(q * LANE, LANE)]
                lo = lax.bitcast_convert_type(
                    lax.shift_left(v, 16), jnp.float32)
                hi = lax.bitcast_convert_type(v & himask, jnp.float32)
                plsc.addupdate_scatter(
                    acc, [cvec, iota16 + (2 * q * LANE)], lo * wv)
                plsc.addupdate_scatter(
                    acc, [cvec, iota16 + ((2 * q + 1) * LANE)], hi * wv)

        @pl.when(k + 2 < nch)
        def _():
            idx_wait(k + 2, isem)
            gather_desc(k + 2, rg, gsem).start()

        @pl.when(k + 4 < nch)
        def _():
            idx_start(k + 4, isem)

    def pair(m, carry):
        k = 2 * m
        slot_step(k, rg0, gsem0, isem0)
        slot_step(k + 1, rg1, gsem1, isem1)
        return carry

    lax.fori_loop(0, npairs, pair, 0)
    pltpu.sync_copy(acc.at[pl.ds(0, RPW)],
                    out_ref.at[pl.ds(pl.multiple_of(wid * RPW, 8), RPW)])


BN = 1000


def _round_bf16_bits(x):
    u = lax.bitcast_convert_type(x, jnp.int32)
    rb = lax.shift_right_logical(u, 16) & 1
    return lax.shift_right_logical(u + 32767 + rb, 16)


def _pack_rows(h):
    bn = h.shape[0]
    h4 = h.reshape(bn, D // 32, 2, LANE)
    lo = _round_bf16_bits(h4[:, :, 0, :])
    hi = _round_bf16_bits(h4[:, :, 1, :])
    return (lo | lax.shift_left(hi, 16)).reshape(bn, D // 2)


def _pack_body(h_ref, ob_ref):
    ob_ref[...] = _pack_rows(h_ref[...])


def _pack(h):
    return pl.pallas_call(
        _pack_body,
        grid=(N // BN,),
        in_specs=[pl.BlockSpec((BN, D), lambda i: (i, 0))],
        out_specs=pl.BlockSpec((BN, D // 2), lambda i: (i, 0)),
        out_shape=jax.ShapeDtypeStruct((N, D // 2), jnp.int32),
    )(h)


def _dense_body(tx0_ref, tx1_ref, q_ref, w_ref, b_ref, o_ref, ob_ref):
    tx0 = tx0_ref[...]
    tx1 = tx1_ref[...]
    tx2 = 2.0 * q_ref[...] - tx0
    acc = jnp.dot(tx0, w_ref[0], preferred_element_type=jnp.float32)
    acc = acc + jnp.dot(tx1, w_ref[1], preferred_element_type=jnp.float32)
    acc = acc + jnp.dot(tx2, w_ref[2], preferred_element_type=jnp.float32)
    h = jax.nn.sigmoid(acc + b_ref[...])
    o_ref[...] = h
    ob_ref[...] = _pack_rows(h)


def _dense(tx0, tx1, q, w, b):
    return pl.pallas_call(
        _dense_body,
        grid=(N // BN,),
        in_specs=[
            pl.BlockSpec((BN, D), lambda i: (i, 0)),
            pl.BlockSpec((BN, D), lambda i: (i, 0)),
            pl.BlockSpec((BN, D), lambda i: (i, 0)),
            pl.BlockSpec((3, D, D), lambda i: (0, 0, 0)),
            pl.BlockSpec((1, D), lambda i: (0, 0)),
        ],
        out_specs=[pl.BlockSpec((BN, D), lambda i: (i, 0)),
                   pl.BlockSpec((BN, D // 2), lambda i: (i, 0))],
        out_shape=[jax.ShapeDtypeStruct((N, D), jnp.float32),
                   jax.ShapeDtypeStruct((N, D // 2), jnp.int32)],
    )(tx0, tx1, q, w, b)


def kernel(x, edge_index, edge_weight, W1, b1, W2, b2, W3, b3, W4, b4):
    perm = jnp.argsort(edge_index[1])
    row = edge_index[0][perm]
    col = edge_index[1][perm]
    ew = edge_weight[perm]
    pad = EPAD - E
    row_p = jnp.pad(row, (0, pad))
    col_p = jnp.pad(col, (0, pad))
    ew_p = jnp.pad(ew, (0, pad))

    o = jnp.searchsorted(col, jnp.arange(NW + 1, dtype=jnp.int32) * RPW)
    o = o.astype(jnp.int32)
    o_t = o[:-1]
    e_t = o[1:]
    s_t = o_t - (o_t % 8)
    nch = (e_t - s_t + PCHUNK - 1) // PCHUNK
    npairs = jnp.maximum((nch + 1) // 2, 1)
    base = jnp.arange(NW, dtype=jnp.int32) * RPW
    zero_col = jnp.zeros((NW,), jnp.int32)
    params = jnp.stack(
        [o_t, e_t, s_t, npairs, base] + [zero_col] * 11, axis=1
    ).reshape(NW * LANE).astype(jnp.int32)

    partials = _deg_partials(row_p, col_p, ew_p).reshape(NW, N)
    dinv = _dinv(partials).reshape(N)
    lw = _lap_weights(row_p, col_p, ew_p, dinv)

    h = x
    hp = _pack(x)
    for w, b in ((W1, b1), (W2, b2), (W3, b3), (W4, b4)):
        p = _prop(row_p, col_p, lw, hp, params)
        tx1p = _pack(p)
        q = _prop(row_p, col_p, lw, tx1p, params)
        h, hp = _dense(h, p, q, w, b.reshape(1, D))
    return h
